# baseline (speedup 1.0000x reference)
import functools

import jax
import jax.numpy as jnp
from jax import lax
from jax.experimental import pallas as pl
from jax.experimental.pallas import tpu as pltpu
from jax.experimental.pallas import tpu_sc as plsc

NC = 2
NS = 16
NW = NC * NS
B = 16384
K = 32
L = 16
BPW = B // NW

NUM_USERS = 100000
NUM_ITEMS = 1000000
ITEM_BLOCKS = NUM_ITEMS // 128
ITEM_TAIL_START = 7812 * 128
MAX_ITEM_BLOCK = 7811

_mesh = plsc.VectorSubcoreMesh(core_axis_name="c", subcore_axis_name="s")


@functools.partial(
    pl.kernel,
    mesh=_mesh,
    out_type=(
        jax.ShapeDtypeStruct((B,), jnp.float32),
        jax.ShapeDtypeStruct((NC, NUM_USERS // 4, 128), jnp.float32),
    ),
    compiler_params=pltpu.CompilerParams(
        needs_layout_passes=False, use_tc_tiling_on_sc=True),
    scratch_types=[
        pltpu.VMEM((4, 128), jnp.int32),
        pltpu.VMEM((4, 128), jnp.int32),
        pltpu.VMEM((4, 128), jnp.int32),
        pltpu.VMEM((16, 32, 128), jnp.float32),
        pltpu.VMEM((16, 128), jnp.float32),
        pltpu.VMEM((1, 128), jnp.float32),
        pltpu.VMEM((8, 128), jnp.float32),
        pltpu.VMEM((128, 128), jnp.float32),
        pltpu.VMEM((32, BPW), jnp.float32),
        pltpu.VMEM((32, BPW), jnp.float32),
        pltpu.VMEM((16, 128), jnp.float32),
        pltpu.VMEM((BPW,), jnp.float32),
        pltpu.VMEM((BPW,), jnp.float32),
        pltpu.VMEM((BPW,), jnp.float32),
        pltpu.SemaphoreType.DMA,
        pltpu.SemaphoreType.DMA,
    ],
)
def _mcf_sc(xt_hbm, ut_hbm, it_hbm, itail_hbm, utail_hbm, ub_hbm, ibt_hbm,
            ibtail_hbm, out_hbm, u4s_hbm, uidx_v, iidx_v, u4idx_v, blk_v,
            bslot_v, ibt_v, utail_v, urows_v, ic_v, uc_v,
            itail_v, ubias_v, ibias_v, acc_v, sem, semb):
    cid = lax.axis_index("c")
    sid = lax.axis_index("s")
    wid = sid * NC + cid
    base = wid * BPW

    pltpu.sync_copy(xt_hbm.at[0].at[pl.ds(wid * 4, 4)], uidx_v)
    pltpu.sync_copy(xt_hbm.at[1].at[pl.ds(wid * 4, 4)], iidx_v)
    pltpu.sync_copy(itail_hbm, itail_v)
    pltpu.sync_copy(ibtail_hbm, ibt_v)
    pltpu.sync_copy(utail_hbm, utail_v)

    def stage_block(g, carry):
        c = g * NS + sid

        @pl.when(c < 781)
        def _do():
            cb = pl.multiple_of(c * 128, 128)
            ubv = blk_v.at[0]
            utv = blk_v.at[1]
            pltpu.async_copy(ut_hbm.at[:, pl.ds(cb, 128)], ubv, sem).wait()
            for k in range(32):
                for q in range(4):
                    for h in range(2):
                        vals = plsc.load_gather(
                            ubv,
                            [h * L + lax.iota(jnp.int32, L),
                             jnp.full((L,), 4 * k + q, jnp.int32)])
                        utv[k, pl.ds(q * K + h * L, L)] = vals
            pltpu.async_copy(
                utv, u4s_hbm.at[cid].at[pl.ds(c * 32, 32)], sem).wait()

        return carry

    lax.fori_loop(0, 49, stage_block, 0)
    plsc.subcore_barrier()

    for k in range(4):
        for c in range(8):
            vec = uidx_v[k, pl.ds(c * L, L)]
            u4idx_v[k, pl.ds(c * L, L)] = vec >> 2

    bias_copies = []
    for k in range(4):
        bias_copies.append(pltpu.async_copy(
            ub_hbm.at[uidx_v.at[k]], ubias_v.at[pl.ds(k * 128, 128)], semb))

    iota = lax.iota(jnp.int32, L)

    def item_group(g, carry):
        ivec = iidx_v[g // 8, pl.ds((g % 8) * L, L)]
        colvec = ivec & 127
        copies = []
        for lane in range(L):
            u = ivec[lane]
            cblk = pl.multiple_of(
                jnp.minimum(u >> 7, MAX_ITEM_BLOCK) * 128, 128)
            copies.append(pltpu.async_copy(
                it_hbm.at[:, pl.ds(cblk, 128)], blk_v.at[lane], sem))
            copies.append(pltpu.async_copy(
                ibt_hbm.at[:, pl.ds(cblk, 128)],
                bslot_v.at[pl.ds(lane, 1)], sem))
        for cp in copies:
            cp.wait()
        j0 = g * L
        for lane in range(L):
            colv = jnp.full((L,), colvec[lane], jnp.int32)
            jv = jnp.full((L,), j0 + lane, jnp.int32)
            for h in range(2):
                vals = plsc.load_gather(
                    blk_v.at[lane], [h * L + iota, colv])
                plsc.store_scatter(ic_v, [h * L + iota, jv], vals)
        bvals = plsc.load_gather(bslot_v, [iota, colvec])
        ibias_v[pl.ds(j0, L)] = bvals
        t = ivec - ITEM_TAIL_START
        is_tail = t >= 0
        ntail = plsc.all_reduce_population_count(is_tail)

        @pl.when(ntail[0] > 0)
        def _fix():
            tc = jnp.clip(t, 0, 63)
            trow = tc >> 2
            tcol0 = (tc & 3) * K
            jvec = j0 + iota
            for d in range(K):
                dvec = jnp.full((L,), d, jnp.int32)
                vals = plsc.load_gather(itail_v, [trow, tcol0 + dvec])
                plsc.store_scatter(ic_v, [dvec, jvec], vals, mask=is_tail)
            tb = plsc.load_gather(ibt_v, [jnp.zeros((L,), jnp.int32), tc])
            plsc.store_scatter(ibias_v, [jvec], tb, mask=is_tail)

        return carry

    lax.fori_loop(0, BPW // L, item_group, 0)

    for k in range(4):
        pltpu.async_copy(
            u4s_hbm.at[cid].at[u4idx_v.at[k]], urows_v, sem).wait()

        def user_group(g2, carry, k=k):
            uvec = uidx_v[k, pl.ds(g2 * L, L)]
            j0 = k * 128 + g2 * L
            for lane in range(L):
                u = uvec[lane]
                rowv = jnp.full((L,), g2 * L + lane, jnp.int32)
                col0 = (u & 3) * K
                jv = jnp.full((L,), j0 + lane, jnp.int32)
                for h in range(2):
                    vals = plsc.load_gather(
                        urows_v, [rowv, col0 + h * L + iota])
                    plsc.store_scatter(uc_v, [h * L + iota, jv], vals)
            t = uvec - (781 * 128)
            is_tail = t >= 0
            ntail = plsc.all_reduce_population_count(is_tail)

            @pl.when(ntail[0] > 0)
            def _fix():
                tc = jnp.clip(t, 0, 31)
                trow = tc >> 2
                tcol0 = (tc & 3) * K
                jvec = j0 + iota
                for d in range(K):
                    dvec = jnp.full((L,), d, jnp.int32)
                    vals = plsc.load_gather(utail_v, [trow, tcol0 + dvec])
                    plsc.store_scatter(uc_v, [dvec, jvec], vals,
                                       mask=is_tail)

            return carry

        lax.fori_loop(0, 8, user_group, 0)

    for cp in bias_copies:
        cp.wait()

    def sweep(g, carry):
        sl = pl.ds(g * L, L)
        acc = ubias_v[sl] + ibias_v[sl]
        for d in range(K):
            acc = acc + jnp.maximum(uc_v[d, sl] * ic_v[d, sl], 0.0)
        acc_v[sl] = acc
        return carry

    lax.fori_loop(0, BPW // L, sweep, 0)

    pltpu.sync_copy(acc_v, out_hbm.at[pl.ds(base, BPW)])


def kernel(x, user_latent, item_latent, user_bias, item_bias):
    xt = x.T.reshape(2, NW * 4, 128)
    ut = user_latent.T
    it = item_latent.T
    itail = lax.dynamic_slice(
        item_latent, (ITEM_TAIL_START, 0), (64, K)).reshape(16, 128)
    utail = lax.dynamic_slice(
        user_latent, (781 * 128, 0), (32, K)).reshape(8, 128)
    ibt = item_bias.T
    ibtail = jnp.pad(
        lax.dynamic_slice(item_bias, (ITEM_TAIL_START, 0), (64, 1))
        .reshape(1, 64), ((0, 0), (0, 64)))
    out, _ = _mcf_sc(xt, ut, it, itail, utail,
                     user_bias.reshape(-1), ibt, ibtail)
    return out.reshape(B, 1)


# speedup vs baseline: 1.7529x; 1.7529x over previous
import functools

import jax
import jax.numpy as jnp
from jax import lax
from jax.experimental import pallas as pl
from jax.experimental.pallas import tpu as pltpu
from jax.experimental.pallas import tpu_sc as plsc

NC = 2
NS = 16
NW = NC * NS
B = 16384
K = 32
L = 16
BPW = B // NW

NUM_USERS = 100000
NUM_ITEMS = 1000000
ITEM_BLOCKS = NUM_ITEMS // 128
ITEM_TAIL_START = 7812 * 128
MAX_ITEM_BLOCK = 7811

_mesh = plsc.VectorSubcoreMesh(core_axis_name="c", subcore_axis_name="s")


@functools.partial(
    pl.kernel,
    mesh=_mesh,
    out_type=jax.ShapeDtypeStruct((B,), jnp.float32),
    compiler_params=pltpu.CompilerParams(
        needs_layout_passes=False, use_tc_tiling_on_sc=True),
    scratch_types=[
        pltpu.VMEM((4, 128), jnp.int32),
        pltpu.VMEM((4, 128), jnp.int32),
        pltpu.VMEM((4, 128), jnp.int32),
        pltpu.VMEM((16, 32, 128), jnp.float32),
        pltpu.VMEM((16, 128), jnp.float32),
        pltpu.VMEM((1, 128), jnp.float32),
        pltpu.VMEM((128, 128), jnp.float32),
        pltpu.VMEM((32, BPW), jnp.float32),
        pltpu.VMEM((32, BPW), jnp.float32),
        pltpu.VMEM((16, 128), jnp.float32),
        pltpu.VMEM((BPW,), jnp.float32),
        pltpu.VMEM((BPW,), jnp.float32),
        pltpu.VMEM((BPW,), jnp.float32),
        pltpu.SemaphoreType.DMA,
        pltpu.SemaphoreType.DMA,
    ],
)
def _mcf_sc(xt_hbm, u4_hbm, it_hbm, itail_hbm, ub_hbm, ibt_hbm, ibtail_hbm,
            out_hbm, uidx_v, iidx_v, u4idx_v, blk_v, bslot_v, ibt_v,
            urows_v, ic_v, uc_v, itail_v, ubias_v, ibias_v, acc_v,
            sem, semb):
    wid = lax.axis_index("s") * NC + lax.axis_index("c")
    base = wid * BPW

    pltpu.sync_copy(xt_hbm.at[0].at[pl.ds(wid * 4, 4)], uidx_v)
    pltpu.sync_copy(xt_hbm.at[1].at[pl.ds(wid * 4, 4)], iidx_v)
    pltpu.sync_copy(itail_hbm, itail_v)
    pltpu.sync_copy(ibtail_hbm, ibt_v)

    for k in range(4):
        for c in range(8):
            vec = uidx_v[k, pl.ds(c * L, L)]
            u4idx_v[k, pl.ds(c * L, L)] = vec >> 2

    bias_copies = []
    for k in range(4):
        bias_copies.append(pltpu.async_copy(
            ub_hbm.at[uidx_v.at[k]], ubias_v.at[pl.ds(k * 128, 128)], semb))

    iota = lax.iota(jnp.int32, L)

    def item_group(g, carry):
        ivec = iidx_v[g // 8, pl.ds((g % 8) * L, L)]
        colvec = ivec & 127
        copies = []
        for lane in range(L):
            u = ivec[lane]
            cblk = pl.multiple_of(
                jnp.minimum(u >> 7, MAX_ITEM_BLOCK) * 128, 128)
            copies.append(pltpu.async_copy(
                it_hbm.at[:, pl.ds(cblk, 128)], blk_v.at[lane], sem))
            copies.append(pltpu.async_copy(
                ibt_hbm.at[:, pl.ds(cblk, 128)],
                bslot_v.at[pl.ds(lane, 1)], sem))
        for cp in copies:
            cp.wait()
        j0 = g * L
        for lane in range(L):
            colv = jnp.full((L,), colvec[lane], jnp.int32)
            jv = jnp.full((L,), j0 + lane, jnp.int32)
            for h in range(2):
                vals = plsc.load_gather(
                    blk_v.at[lane], [h * L + iota, colv])
                plsc.store_scatter(ic_v, [h * L + iota, jv], vals)
        bvals = plsc.load_gather(bslot_v, [iota, colvec])
        ibias_v[pl.ds(j0, L)] = bvals
        t = ivec - ITEM_TAIL_START
        is_tail = t >= 0
        ntail = plsc.all_reduce_population_count(is_tail)

        @pl.when(ntail[0] > 0)
        def _fix():
            tc = jnp.clip(t, 0, 63)
            trow = tc >> 2
            tcol0 = (tc & 3) * K
            jvec = j0 + iota
            for d in range(K):
                dvec = jnp.full((L,), d, jnp.int32)
                vals = plsc.load_gather(itail_v, [trow, tcol0 + dvec])
                plsc.store_scatter(ic_v, [dvec, jvec], vals, mask=is_tail)
            tb = plsc.load_gather(ibt_v, [jnp.zeros((L,), jnp.int32), tc])
            plsc.store_scatter(ibias_v, [jvec], tb, mask=is_tail)

        return carry

    lax.fori_loop(0, BPW // L, item_group, 0)

    for k in range(4):
        pltpu.async_copy(u4_hbm.at[u4idx_v.at[k]], urows_v, sem).wait()

        def user_group(g2, carry, k=k):
            uvec = uidx_v[k, pl.ds(g2 * L, L)]
            j0 = k * 128 + g2 * L
            for lane in range(L):
                u = uvec[lane]
                rowv = jnp.full((L,), g2 * L + lane, jnp.int32)
                col0 = (u & 3) * K
                jv = jnp.full((L,), j0 + lane, jnp.int32)
                for h in range(2):
                    vals = plsc.load_gather(
                        urows_v, [rowv, col0 + h * L + iota])
                    plsc.store_scatter(uc_v, [h * L + iota, jv], vals)
            return carry

        lax.fori_loop(0, 8, user_group, 0)

    for cp in bias_copies:
        cp.wait()

    def sweep(g, carry):
        sl = pl.ds(g * L, L)
        acc = ubias_v[sl] + ibias_v[sl]
        for d in range(K):
            acc = acc + jnp.maximum(uc_v[d, sl] * ic_v[d, sl], 0.0)
        acc_v[sl] = acc
        return carry

    lax.fori_loop(0, BPW // L, sweep, 0)

    pltpu.sync_copy(acc_v, out_hbm.at[pl.ds(base, BPW)])


def kernel(x, user_latent, item_latent, user_bias, item_bias):
    xt = x.T.reshape(2, NW * 4, 128)
    u4 = user_latent.reshape(NUM_USERS // 4, 128)
    it = item_latent.T
    itail = lax.dynamic_slice(
        item_latent, (ITEM_TAIL_START, 0), (64, K)).reshape(16, 128)
    ibt = item_bias.T
    ibtail = jnp.pad(
        lax.dynamic_slice(item_bias, (ITEM_TAIL_START, 0), (64, 1))
        .reshape(1, 64), ((0, 0), (0, 64)))
    out = _mcf_sc(xt, u4, it, itail,
                  user_bias.reshape(-1), ibt, ibtail)
    return out.reshape(B, 1)


# speedup vs baseline: 1.8908x; 1.0786x over previous
import functools

import jax
import jax.numpy as jnp
from jax import lax
from jax.experimental import pallas as pl
from jax.experimental.pallas import tpu as pltpu
from jax.experimental.pallas import tpu_sc as plsc

NC = 2
NS = 16
NW = NC * NS
B = 16384
K = 32
L = 16
BPW = B // NW

NUM_USERS = 100000
NUM_ITEMS = 1000000
ITEM_TAIL_START = 7812 * 128
MAX_ITEM_BLOCK = 7811

_mesh = plsc.VectorSubcoreMesh(core_axis_name="c", subcore_axis_name="s")
_params = pltpu.CompilerParams(
    needs_layout_passes=False, use_tc_tiling_on_sc=True)


@functools.partial(
    pl.kernel,
    mesh=_mesh,
    out_type=(
        jax.ShapeDtypeStruct((32, B), jnp.float32),
        jax.ShapeDtypeStruct((B,), jnp.float32),
    ),
    compiler_params=_params,
    scratch_types=[
        pltpu.VMEM((4, 128), jnp.int32),
        pltpu.VMEM((16, 32, 128), jnp.float32),
        pltpu.VMEM((16, 128), jnp.float32),
        pltpu.VMEM((1, 128), jnp.float32),
        pltpu.VMEM((32, BPW), jnp.float32),
        pltpu.VMEM((16, 128), jnp.float32),
        pltpu.VMEM((BPW,), jnp.float32),
        pltpu.SemaphoreType.DMA,
    ],
)
def _mcf_item(xt_hbm, it_hbm, itail_hbm, ibt_hbm, ibtail_hbm,
              ic_hbm, ibias_hbm, iidx_v, blk_v, bslot_v, ibt_v,
              ic_v, itail_v, ibias_v, sem):
    wid = lax.axis_index("s") * NC + lax.axis_index("c")
    base = wid * BPW

    pltpu.sync_copy(xt_hbm.at[1].at[pl.ds(wid * 4, 4)], iidx_v)
    pltpu.sync_copy(itail_hbm, itail_v)
    pltpu.sync_copy(ibtail_hbm, ibt_v)

    iota = lax.iota(jnp.int32, L)

    def item_group(g, carry):
        ivec = iidx_v[g // 8, pl.ds((g % 8) * L, L)]
        colvec = ivec & 127
        copies = []
        for lane in range(L):
            u = ivec[lane]
            cblk = pl.multiple_of(
                jnp.minimum(u >> 7, MAX_ITEM_BLOCK) * 128, 128)
            copies.append(pltpu.async_copy(
                it_hbm.at[:, pl.ds(cblk, 128)], blk_v.at[lane], sem))
            copies.append(pltpu.async_copy(
                ibt_hbm.at[:, pl.ds(cblk, 128)],
                bslot_v.at[pl.ds(lane, 1)], sem))
        for cp in copies:
            cp.wait()
        j0 = g * L
        for lane in range(L):
            colv = jnp.full((L,), colvec[lane], jnp.int32)
            jv = jnp.full((L,), j0 + lane, jnp.int32)
            for h in range(2):
                vals = plsc.load_gather(
                    blk_v.at[lane], [h * L + iota, colv])
                plsc.store_scatter(ic_v, [h * L + iota, jv], vals)
        bvals = plsc.load_gather(bslot_v, [iota, colvec])
        ibias_v[pl.ds(j0, L)] = bvals
        t = ivec - ITEM_TAIL_START
        is_tail = t >= 0
        ntail = plsc.all_reduce_population_count(is_tail)

        @pl.when(ntail[0] > 0)
        def _fix():
            tc = jnp.clip(t, 0, 63)
            trow = tc >> 2
            tcol0 = (tc & 3) * K
            jvec = j0 + iota
            for d in range(K):
                dvec = jnp.full((L,), d, jnp.int32)
                vals = plsc.load_gather(itail_v, [trow, tcol0 + dvec])
                plsc.store_scatter(ic_v, [dvec, jvec], vals, mask=is_tail)
            tb = plsc.load_gather(ibt_v, [jnp.zeros((L,), jnp.int32), tc])
            plsc.store_scatter(ibias_v, [jvec], tb, mask=is_tail)

        return carry

    lax.fori_loop(0, BPW // L, item_group, 0)

    pltpu.sync_copy(ic_v, ic_hbm.at[:, pl.ds(base, BPW)])
    pltpu.sync_copy(ibias_v, ibias_hbm.at[pl.ds(base, BPW)])


@functools.partial(
    pl.kernel,
    mesh=_mesh,
    out_type=jax.ShapeDtypeStruct((B,), jnp.float32),
    compiler_params=_params,
    scratch_types=[
        pltpu.VMEM((4, 128), jnp.int32),
        pltpu.VMEM((4, 128), jnp.int32),
        pltpu.VMEM((128, 128), jnp.float32),
        pltpu.VMEM((32, BPW), jnp.float32),
        pltpu.VMEM((32, BPW), jnp.float32),
        pltpu.VMEM((BPW,), jnp.float32),
        pltpu.VMEM((BPW,), jnp.float32),
        pltpu.VMEM((BPW,), jnp.float32),
        pltpu.SemaphoreType.DMA,
        pltpu.SemaphoreType.DMA,
    ],
)
def _mcf_user(xt_hbm, u4_hbm, ub_hbm, ic_hbm, ibias_hbm,
              out_hbm, uidx_v, u4idx_v, urows_v, uc_v, ic_v,
              ubias_v, ibias_v, acc_v, sem, semb):
    wid = lax.axis_index("s") * NC + lax.axis_index("c")
    base = wid * BPW

    pltpu.sync_copy(xt_hbm.at[0].at[pl.ds(wid * 4, 4)], uidx_v)
    ic_cp = pltpu.async_copy(ic_hbm.at[:, pl.ds(base, BPW)], ic_v, semb)
    ib_cp = pltpu.async_copy(ibias_hbm.at[pl.ds(base, BPW)], ibias_v, semb)

    for k in range(4):
        for c in range(8):
            vec = uidx_v[k, pl.ds(c * L, L)]
            u4idx_v[k, pl.ds(c * L, L)] = vec >> 2

    bias_copies = []
    for k in range(4):
        bias_copies.append(pltpu.async_copy(
            ub_hbm.at[uidx_v.at[k]], ubias_v.at[pl.ds(k * 128, 128)], semb))

    iota = lax.iota(jnp.int32, L)

    for k in range(4):
        pltpu.async_copy(u4_hbm.at[u4idx_v.at[k]], urows_v, sem).wait()

        def user_group(g2, carry, k=k):
            uvec = uidx_v[k, pl.ds(g2 * L, L)]
            j0 = k * 128 + g2 * L
            for lane in range(L):
                u = uvec[lane]
                rowv = jnp.full((L,), g2 * L + lane, jnp.int32)
                col0 = (u & 3) * K
                jv = jnp.full((L,), j0 + lane, jnp.int32)
                for h in range(2):
                    vals = plsc.load_gather(
                        urows_v, [rowv, col0 + h * L + iota])
                    plsc.store_scatter(uc_v, [h * L + iota, jv], vals)
            return carry

        lax.fori_loop(0, 8, user_group, 0)

    for cp in bias_copies:
        cp.wait()
    ic_cp.wait()
    ib_cp.wait()

    def sweep(g, carry):
        sl = pl.ds(g * L, L)
        acc = ubias_v[sl] + ibias_v[sl]
        for d in range(K):
            acc = acc + jnp.maximum(uc_v[d, sl] * ic_v[d, sl], 0.0)
        acc_v[sl] = acc
        return carry

    lax.fori_loop(0, BPW // L, sweep, 0)

    pltpu.sync_copy(acc_v, out_hbm.at[pl.ds(base, BPW)])


def kernel(x, user_latent, item_latent, user_bias, item_bias):
    xt = x.T.reshape(2, NW * 4, 128)
    u4 = user_latent.reshape(NUM_USERS // 4, 128)
    it = item_latent.T
    itail = lax.dynamic_slice(
        item_latent, (ITEM_TAIL_START, 0), (64, K)).reshape(16, 128)
    ibt = item_bias.T
    ibtail = jnp.pad(
        lax.dynamic_slice(item_bias, (ITEM_TAIL_START, 0), (64, 1))
        .reshape(1, 64), ((0, 0), (0, 64)))
    ic, ibias = _mcf_item(xt, it, itail, ibt, ibtail)
    out = _mcf_user(xt, u4, user_bias.reshape(-1), ic, ibias)
    return out.reshape(B, 1)


# speedup vs baseline: 1.9288x; 1.0201x over previous
import functools

import jax
import jax.numpy as jnp
from jax import lax
from jax.experimental import pallas as pl
from jax.experimental.pallas import tpu as pltpu
from jax.experimental.pallas import tpu_sc as plsc

NC = 2
NS = 16
NW = NC * NS
B = 16384
K = 32
L = 16
BPW = B // NW

NUM_USERS = 100000
NUM_ITEMS = 1000000
ITEM_TAIL_START = 7812 * 128
MAX_ITEM_BLOCK = 7811

_mesh = plsc.VectorSubcoreMesh(core_axis_name="c", subcore_axis_name="s")
_params = pltpu.CompilerParams(
    needs_layout_passes=False, use_tc_tiling_on_sc=True)


@functools.partial(
    pl.kernel,
    mesh=_mesh,
    out_type=(
        jax.ShapeDtypeStruct((32, B), jnp.float32),
        jax.ShapeDtypeStruct((B,), jnp.float32),
    ),
    compiler_params=_params,
    scratch_types=[
        pltpu.VMEM((4, 128), jnp.int32),
        pltpu.VMEM((16, 32, 128), jnp.float32),
        pltpu.VMEM((16, 128), jnp.float32),
        pltpu.VMEM((1, 128), jnp.float32),
        pltpu.VMEM((32, BPW), jnp.float32),
        pltpu.VMEM((16, 128), jnp.float32),
        pltpu.VMEM((BPW,), jnp.float32),
        pltpu.SemaphoreType.DMA,
    ],
)
def _mcf_item(xt_hbm, it_hbm, itail_hbm, ibt_hbm, ibtail_hbm,
              ic_hbm, ibias_hbm, iidx_v, blk_v, bslot_v, ibt_v,
              ic_v, itail_v, ibias_v, sem):
    wid = lax.axis_index("s") * NC + lax.axis_index("c")
    base = wid * BPW

    pltpu.sync_copy(xt_hbm.at[1].at[pl.ds(wid * 4, 4)], iidx_v)
    pltpu.sync_copy(itail_hbm, itail_v)
    pltpu.sync_copy(ibtail_hbm, ibt_v)

    iota = lax.iota(jnp.int32, L)

    def item_group(g, carry):
        ivec = iidx_v[g // 8, pl.ds((g % 8) * L, L)]
        colvec = ivec & 127
        copies = []
        for lane in range(L):
            u = ivec[lane]
            cblk = pl.multiple_of(
                jnp.minimum(u >> 7, MAX_ITEM_BLOCK) * 128, 128)
            copies.append(pltpu.async_copy(
                it_hbm.at[:, pl.ds(cblk, 128)], blk_v.at[lane], sem))
            copies.append(pltpu.async_copy(
                ibt_hbm.at[:, pl.ds(cblk, 128)],
                bslot_v.at[pl.ds(lane, 1)], sem))
        for cp in copies:
            cp.wait()
        j0 = g * L
        for lane in range(L):
            colv = jnp.full((L,), colvec[lane], jnp.int32)
            jv = jnp.full((L,), j0 + lane, jnp.int32)
            for h in range(2):
                vals = plsc.load_gather(
                    blk_v.at[lane], [h * L + iota, colv])
                plsc.store_scatter(ic_v, [h * L + iota, jv], vals)
        bvals = plsc.load_gather(bslot_v, [iota, colvec])
        ibias_v[pl.ds(j0, L)] = bvals
        t = ivec - ITEM_TAIL_START
        is_tail = t >= 0
        ntail = plsc.all_reduce_population_count(is_tail)

        @pl.when(ntail[0] > 0)
        def _fix():
            tc = jnp.clip(t, 0, 63)
            trow = tc >> 2
            tcol0 = (tc & 3) * K
            jvec = j0 + iota
            for d in range(K):
                dvec = jnp.full((L,), d, jnp.int32)
                vals = plsc.load_gather(itail_v, [trow, tcol0 + dvec])
                plsc.store_scatter(ic_v, [dvec, jvec], vals, mask=is_tail)
            tb = plsc.load_gather(ibt_v, [jnp.zeros((L,), jnp.int32), tc])
            plsc.store_scatter(ibias_v, [jvec], tb, mask=is_tail)

        return carry

    lax.fori_loop(0, BPW // L, item_group, 0)

    pltpu.sync_copy(ic_v, ic_hbm.at[:, pl.ds(base, BPW)])
    pltpu.sync_copy(ibias_v, ibias_hbm.at[pl.ds(base, BPW)])


@functools.partial(
    pl.kernel,
    mesh=_mesh,
    out_type=jax.ShapeDtypeStruct((B,), jnp.float32),
    compiler_params=_params,
    scratch_types=[
        pltpu.VMEM((4, 128), jnp.int32),
        pltpu.VMEM((4, 128), jnp.int32),
        pltpu.VMEM((2, 128, 128), jnp.float32),
        pltpu.VMEM((32, BPW), jnp.float32),
        pltpu.VMEM((32, BPW), jnp.float32),
        pltpu.VMEM((BPW,), jnp.float32),
        pltpu.VMEM((BPW,), jnp.float32),
        pltpu.VMEM((BPW,), jnp.float32),
        pltpu.SemaphoreType.DMA,
        pltpu.SemaphoreType.DMA,
    ],
)
def _mcf_user(xt_hbm, u4_hbm, ub_hbm, ic_hbm, ibias_hbm,
              out_hbm, uidx_v, u4idx_v, urows_v, uc_v, ic_v,
              ubias_v, ibias_v, acc_v, sem, semb):
    wid = lax.axis_index("s") * NC + lax.axis_index("c")
    base = wid * BPW

    pltpu.sync_copy(xt_hbm.at[0].at[pl.ds(wid * 4, 4)], uidx_v)
    ic_cp = pltpu.async_copy(ic_hbm.at[:, pl.ds(base, BPW)], ic_v, semb)
    ib_cp = pltpu.async_copy(ibias_hbm.at[pl.ds(base, BPW)], ibias_v, semb)

    for k in range(4):
        for c in range(8):
            vec = uidx_v[k, pl.ds(c * L, L)]
            u4idx_v[k, pl.ds(c * L, L)] = vec >> 2

    bias_copies = []
    for k in range(4):
        bias_copies.append(pltpu.async_copy(
            ub_hbm.at[uidx_v.at[k]], ubias_v.at[pl.ds(k * 128, 128)], semb))

    iota = lax.iota(jnp.int32, L)

    chunk_cps = [pltpu.async_copy(
        u4_hbm.at[u4idx_v.at[k]], urows_v.at[k % 2], sem)
        for k in range(2)]
    for k in range(4):
        chunk_cps[k].wait()

        def user_group(g2, carry, k=k):
            uvec = uidx_v[k, pl.ds(g2 * L, L)]
            j0 = k * 128 + g2 * L
            for lane in range(L):
                u = uvec[lane]
                rowv = jnp.full((L,), g2 * L + lane, jnp.int32)
                col0 = (u & 3) * K
                jv = jnp.full((L,), j0 + lane, jnp.int32)
                for h in range(2):
                    vals = plsc.load_gather(
                        urows_v.at[k % 2], [rowv, col0 + h * L + iota])
                    plsc.store_scatter(uc_v, [h * L + iota, jv], vals)
            return carry

        lax.fori_loop(0, 8, user_group, 0)
        if k + 2 < 4:
            chunk_cps.append(pltpu.async_copy(
                u4_hbm.at[u4idx_v.at[k + 2]], urows_v.at[k % 2], sem))

    for cp in bias_copies:
        cp.wait()
    ic_cp.wait()
    ib_cp.wait()

    def sweep(g, carry):
        sl = pl.ds(g * L, L)
        acc = ubias_v[sl] + ibias_v[sl]
        for d in range(K):
            acc = acc + jnp.maximum(uc_v[d, sl] * ic_v[d, sl], 0.0)
        acc_v[sl] = acc
        return carry

    lax.fori_loop(0, BPW // L, sweep, 0)

    pltpu.sync_copy(acc_v, out_hbm.at[pl.ds(base, BPW)])


def kernel(x, user_latent, item_latent, user_bias, item_bias):
    xt = x.T.reshape(2, NW * 4, 128)
    u4 = user_latent.reshape(NUM_USERS // 4, 128)
    it = item_latent.T
    itail = lax.dynamic_slice(
        item_latent, (ITEM_TAIL_START, 0), (64, K)).reshape(16, 128)
    ibt = item_bias.T
    ibtail = jnp.pad(
        lax.dynamic_slice(item_bias, (ITEM_TAIL_START, 0), (64, 1))
        .reshape(1, 64), ((0, 0), (0, 64)))
    ic, ibias = _mcf_item(xt, it, itail, ibt, ibtail)
    out = _mcf_user(xt, u4, user_bias.reshape(-1), ic, ibias)
    return out.reshape(B, 1)
